# Initial kernel scaffold; baseline (speedup 1.0000x reference)
import jax
import jax.numpy as jnp
from jax.experimental import pallas as pl


def kernel(f_x, dead_latents):
    raise NotImplementedError("write your pallas kernel here")


# speedup vs baseline: 4.6774x; 4.6774x over previous
import numpy as np
import jax
import jax.numpy as jnp
from jax import lax
from jax.experimental import pallas as pl
from jax.experimental.pallas import tpu as pltpu
from jax.experimental.pallas import tpu_sc as plsc

_TOP_K = 512
_NC, _NS, _L = 2, 16, 16
_NW = _NC * _NS
_NB = 256
_HIST = _L * _NB


def _sc_body(f_hbm, d_hbm, out_hbm, fbuf, dbuf, ubuf, hist):
    B, D = f_hbm.shape
    NV = D // _L
    rows_per_w = B // _NW
    wid = lax.axis_index("s") * _NC + lax.axis_index("c")
    base = wid * rows_per_w
    laneseq = lax.iota(jnp.int32, _L)
    laneoff = laneseq * _NB
    ones = jnp.ones((_L,), jnp.int32)

    def row_step(r, _):
        row = base + r
        pltpu.sync_copy(f_hbm.at[row], fbuf)
        pltpu.sync_copy(d_hbm.at[row], dbuf)

        def prep(i, _):
            s = pl.ds(i * _L, _L)
            f = fbuf[s]
            dd = dbuf[s]
            p = f * dd
            m = p * dd
            bits = lax.bitcast_convert_type(m, jnp.int32)
            u = bits ^ ((bits >> 31) | jnp.int32(-2147483648))
            fbuf[s] = p
            ubuf[s] = lax.bitcast_convert_type(u, jnp.uint32)
            return 0

        lax.fori_loop(0, NV, prep, 0)

        prefix = jnp.uint32(0)
        n_cur = jnp.int32(D)
        k_cur = jnp.int32(_TOP_K)

        for pno in range(4):
            sh = np.uint32(24 - 8 * pno)

            def zero(i, _):
                hist[pl.ds(i * _L, _L)] = jnp.zeros((_L,), jnp.int32)
                return 0

            lax.fori_loop(0, _HIST // _L, zero, 0)

            if pno == 0:
                def scat(i, _):
                    u = ubuf[pl.ds(i * _L, _L)]
                    dig = lax.convert_element_type(
                        (u >> sh) & np.uint32(0xFF), jnp.int32)
                    plsc.addupdate_scatter(hist, [laneoff + dig], ones)
                    return 0
            else:
                psh = np.uint32(32 - 8 * pno)

                def scat(i, _, psh=psh, sh=sh, prefix=prefix):
                    u = ubuf[pl.ds(i * _L, _L)]
                    msk = (u >> psh) == prefix
                    dig = lax.convert_element_type(
                        (u >> sh) & np.uint32(0xFF), jnp.int32)
                    plsc.addupdate_scatter(hist, [laneoff + dig], ones,
                                           mask=msk)
                    return 0

            lax.fori_loop(0, NV, scat, 0)

            C = n_cur - k_cur

            def scan(j, carry, C=C):
                run, bsel, nin, nbelow = carry
                acc = jnp.zeros((_L,), jnp.int32)
                for l in range(_L):
                    acc = acc + hist[pl.ds(l * _NB + j * _L, _L)]
                cum = jnp.cumsum(acc)
                inc = run + cum
                m = inc > C
                s = jnp.sum(m.astype(jnp.int32))
                lane = _L - s
                sel = laneseq == lane
                cnt_at = jnp.sum(jnp.where(sel, acc, 0))
                cum_at = jnp.sum(jnp.where(sel, cum, 0))
                below_at = run + cum_at - cnt_at
                first = jnp.logical_and(s > 0, bsel < jnp.int32(0))
                bsel = jnp.where(first, j * _L + lane, bsel)
                nin = jnp.where(first, cnt_at, nin)
                nbelow = jnp.where(first, below_at, nbelow)
                run = run + jnp.sum(acc)
                return run, bsel, nin, nbelow

            init = (jnp.int32(0), jnp.int32(-1), jnp.int32(0), jnp.int32(0))
            _, bsel, nin, nbelow = lax.fori_loop(0, _NB // _L, scan, init)
            above = n_cur - nbelow - nin
            k_cur = k_cur - above
            n_cur = nin
            prefix = (prefix << np.uint32(8)) | lax.convert_element_type(
                bsel, jnp.uint32)

        thresh = prefix

        def outp(i, _):
            s = pl.ds(i * _L, _L)
            u = ubuf[s]
            p = fbuf[s]
            fbuf[s] = jnp.where(u >= thresh, p, jnp.float32(0.0))
            return 0

        lax.fori_loop(0, NV, outp, 0)

        pltpu.sync_copy(fbuf, out_hbm.at[row])
        return 0

    lax.fori_loop(0, rows_per_w, row_step, 0)


def kernel(f_x, dead_latents):
    B, D = f_x.shape
    mesh = plsc.VectorSubcoreMesh(core_axis_name="c", subcore_axis_name="s",
                                  num_cores=_NC, num_subcores=_NS)
    run = pl.kernel(
        _sc_body,
        out_type=jax.ShapeDtypeStruct((B, D), jnp.float32),
        mesh=mesh,
        compiler_params=pltpu.CompilerParams(needs_layout_passes=False),
        scratch_types=[
            pltpu.VMEM((D,), jnp.float32),
            pltpu.VMEM((D,), jnp.float32),
            pltpu.VMEM((D,), jnp.uint32),
            pltpu.VMEM((_HIST,), jnp.int32),
        ],
    )
    return run(f_x, dead_latents)


# speedup vs baseline: 7.0908x; 1.5160x over previous
import numpy as np
import jax
import jax.numpy as jnp
from jax import lax
from jax.experimental import pallas as pl
from jax.experimental.pallas import tpu as pltpu
from jax.experimental.pallas import tpu_sc as plsc

_TOP_K = 512
_NC, _NS, _L = 2, 16, 16
_NW = _NC * _NS
_NB = 256
_HIST = _L * _NB


def _sc_body(f_hbm, d_hbm, out_hbm, fbuf, dbuf, ubuf, cbuf, hist):
    B, D = f_hbm.shape
    NV = D // _L
    rows_per_w = B // _NW
    wid = lax.axis_index("s") * _NC + lax.axis_index("c")
    base = wid * rows_per_w
    laneseq = lax.iota(jnp.int32, _L)
    laneoff = laneseq * _NB
    ones = jnp.ones((_L,), jnp.int32)
    zeros_v = jnp.zeros((_L,), jnp.int32)
    v15 = jnp.full((_L,), _L - 1, jnp.int32)

    def zero(i, _):
        hist[pl.ds(i * _L, _L)] = zeros_v
        return 0

    lax.fori_loop(0, _HIST // _L, zero, 0)

    def scan_pass(C_v):
        def scan(j, carry):
            run, bsel, nin, nbelow = carry
            acc = zeros_v
            for l in range(_L):
                s = pl.ds(l * _NB + j * _L, _L)
                acc = acc + hist[s]
                hist[s] = zeros_v
            cum = jnp.cumsum(acc)
            inc = run + cum
            m = inc > C_v
            cnt = plsc.all_reduce_population_count(m)
            lane = _L - cnt
            lane_c = jnp.minimum(lane, v15)
            cnt_at = jnp.take_along_axis(acc, lane_c, axis=0)
            cum_at = jnp.take_along_axis(cum, lane_c, axis=0)
            first = jnp.logical_and(cnt > 0, bsel < 0)
            bsel = jnp.where(first, lane + j * _L, bsel)
            nin = jnp.where(first, cnt_at, nin)
            nbelow = jnp.where(first, run + cum_at - cnt_at, nbelow)
            run = run + jnp.take_along_axis(cum, v15, axis=0)
            return run, bsel, nin, nbelow

        init = (zeros_v, jnp.full((_L,), -1, jnp.int32), zeros_v, zeros_v)
        _, bsel, nin, nbelow = lax.fori_loop(0, _NB // _L, scan, init)
        return bsel, nin, nbelow

    def row_step(r, _):
        row = base + r
        pltpu.sync_copy(f_hbm.at[row], fbuf)
        pltpu.sync_copy(d_hbm.at[row], dbuf)

        def prep_scat1(i, _):
            s = pl.ds(i * _L, _L)
            f = fbuf[s]
            dd = dbuf[s]
            p = f * dd
            m = p * dd
            bits = lax.bitcast_convert_type(m, jnp.int32)
            ui = bits ^ ((bits >> 31) | jnp.int32(-2147483648))
            u = lax.bitcast_convert_type(ui, jnp.uint32)
            fbuf[s] = p
            ubuf[s] = u
            dig = lax.convert_element_type(u >> np.uint32(24), jnp.int32)
            plsc.addupdate_scatter(hist, [laneoff + dig], ones)
            return 0

        lax.fori_loop(0, NV, prep_scat1, 0, unroll=4)

        n_cur = jnp.full((_L,), D, jnp.int32)
        k_cur = jnp.full((_L,), _TOP_K, jnp.int32)

        bsel, nin, nbelow = scan_pass(n_cur - k_cur)
        k_cur = k_cur - (n_cur - nbelow - nin)
        n_cur = nin
        prefix = lax.convert_element_type(bsel, jnp.uint32)

        def scat2(i, off, prefix=prefix):
            s = pl.ds(i * _L, _L)
            u = ubuf[s]
            msk = (u >> np.uint32(24)) == prefix
            dig = lax.convert_element_type(
                (u >> np.uint32(16)) & np.uint32(0xFF), jnp.int32)
            plsc.addupdate_scatter(hist, [laneoff + dig], ones, mask=msk)
            plsc.store_compressed(cbuf.at[pl.ds(off, _L)], u, mask=msk)
            return off + jnp.sum(msk.astype(jnp.int32))

        lax.fori_loop(0, NV, scat2, jnp.int32(0), unroll=4)
        n1_s = jnp.max(nin)
        nv1 = (n1_s + _L - 1) // _L
        n1_v = nin

        bsel, nin, nbelow = scan_pass(n_cur - k_cur)
        k_cur = k_cur - (n_cur - nbelow - nin)
        n_cur = nin
        prefix = (prefix << np.uint32(8)) | lax.convert_element_type(
            bsel, jnp.uint32)

        def scat3(j, _, prefix=prefix, n1_v=n1_v):
            s = pl.ds(j * _L, _L)
            u = cbuf[s]
            valid = (laneseq + j * _L) < n1_v
            msk = jnp.logical_and(valid, (u >> np.uint32(16)) == prefix)
            dig = lax.convert_element_type(
                (u >> np.uint32(8)) & np.uint32(0xFF), jnp.int32)
            plsc.addupdate_scatter(hist, [laneoff + dig], ones, mask=msk)
            return 0

        lax.fori_loop(0, nv1, scat3, 0)

        bsel, nin, nbelow = scan_pass(n_cur - k_cur)
        k_cur = k_cur - (n_cur - nbelow - nin)
        n_cur = nin
        prefix = (prefix << np.uint32(8)) | lax.convert_element_type(
            bsel, jnp.uint32)

        def scat4(j, _, prefix=prefix, n1_v=n1_v):
            s = pl.ds(j * _L, _L)
            u = cbuf[s]
            valid = (laneseq + j * _L) < n1_v
            msk = jnp.logical_and(valid, (u >> np.uint32(8)) == prefix)
            dig = lax.convert_element_type(u & np.uint32(0xFF), jnp.int32)
            plsc.addupdate_scatter(hist, [laneoff + dig], ones, mask=msk)
            return 0

        lax.fori_loop(0, nv1, scat4, 0)

        bsel, _, _ = scan_pass(n_cur - k_cur)
        thresh = (prefix << np.uint32(8)) | lax.convert_element_type(
            bsel, jnp.uint32)

        def outp(i, _, thresh=thresh):
            s = pl.ds(i * _L, _L)
            u = ubuf[s]
            p = fbuf[s]
            fbuf[s] = jnp.where(u >= thresh, p, jnp.float32(0.0))
            return 0

        lax.fori_loop(0, NV, outp, 0, unroll=4)

        pltpu.sync_copy(fbuf, out_hbm.at[row])
        return 0

    lax.fori_loop(0, rows_per_w, row_step, 0)


def kernel(f_x, dead_latents):
    B, D = f_x.shape
    mesh = plsc.VectorSubcoreMesh(core_axis_name="c", subcore_axis_name="s",
                                  num_cores=_NC, num_subcores=_NS)
    run = pl.kernel(
        _sc_body,
        out_type=jax.ShapeDtypeStruct((B, D), jnp.float32),
        mesh=mesh,
        compiler_params=pltpu.CompilerParams(needs_layout_passes=False),
        scratch_types=[
            pltpu.VMEM((D,), jnp.float32),
            pltpu.VMEM((D,), jnp.float32),
            pltpu.VMEM((D,), jnp.uint32),
            pltpu.VMEM((D + _L,), jnp.uint32),
            pltpu.VMEM((_HIST,), jnp.int32),
        ],
    )
    return run(f_x, dead_latents)


# speedup vs baseline: 21.5079x; 3.0332x over previous
import numpy as np
import jax
import jax.numpy as jnp
from jax import lax
from jax.experimental import pallas as pl
from jax.experimental.pallas import tpu as pltpu
from jax.experimental.pallas import tpu_sc as plsc

_TOP_K = 512
_NC, _NS, _L = 2, 16, 16
_NW = _NC * _NS
_NB = 256
_HIST = _L * _NB


def _sc_body(f_hbm, d_hbm, out_hbm, fbuf, dbuf, ubuf, cbuf, hist):
    B, D = f_hbm.shape
    rows_per_w = B // _NW
    wid = lax.axis_index("s") * _NC + lax.axis_index("c")
    base = wid * rows_per_w
    laneseq = lax.iota(jnp.int32, _L)
    laneoff = laneseq * _NB
    ones = jnp.ones((_L,), jnp.int32)
    zeros_v = jnp.zeros((_L,), jnp.int32)
    v15 = jnp.full((_L,), _L - 1, jnp.int32)

    @plsc.parallel_loop(0, _HIST, step=_L)
    def _(i):
        hist[pl.ds(i, _L)] = zeros_v

    def scan_pass(C_v):
        init = (zeros_v, jnp.full((_L,), -1, jnp.int32), zeros_v, zeros_v)

        @plsc.parallel_loop(0, _NB, step=_L, carry=init)
        def scan(j, carry):
            run, bsel, nin, nbelow = carry
            acc = zeros_v
            for l in range(_L):
                s = pl.ds(l * _NB + j, _L)
                acc = acc + hist[s]
                hist[s] = zeros_v
            cum = jnp.cumsum(acc)
            inc = run + cum
            m = inc > C_v
            cnt = plsc.all_reduce_population_count(m)
            lane = _L - cnt
            lane_c = jnp.minimum(lane, v15)
            cnt_at = jnp.take_along_axis(acc, lane_c, axis=0)
            cum_at = jnp.take_along_axis(cum, lane_c, axis=0)
            first = jnp.logical_and(cnt > 0, bsel < 0)
            bsel = jnp.where(first, lane + j, bsel)
            nin = jnp.where(first, cnt_at, nin)
            nbelow = jnp.where(first, run + cum_at - cnt_at, nbelow)
            run = run + jnp.take_along_axis(cum, v15, axis=0)
            return run, bsel, nin, nbelow

        _, bsel, nin, nbelow = scan
        return bsel, nin, nbelow

    def row_step(r, _):
        row = base + r
        pltpu.sync_copy(f_hbm.at[row], fbuf)
        pltpu.sync_copy(d_hbm.at[row], dbuf)

        @plsc.parallel_loop(0, D, step=_L, unroll=4)
        def _(i):
            s = pl.ds(i, _L)
            f = fbuf[s]
            dd = dbuf[s]
            m = (f * dd) * dd
            bits = lax.bitcast_convert_type(m, jnp.int32)
            ui = bits ^ ((bits >> 31) | jnp.int32(-2147483648))
            u = lax.bitcast_convert_type(ui, jnp.uint32)
            ubuf[s] = u
            dig = lax.convert_element_type(u >> np.uint32(24), jnp.int32)
            plsc.addupdate_scatter(hist, [laneoff + dig], ones)

        n_cur = jnp.full((_L,), D, jnp.int32)
        k_cur = jnp.full((_L,), _TOP_K, jnp.int32)

        bsel, nin, nbelow = scan_pass(n_cur - k_cur)
        k_cur = k_cur - (n_cur - nbelow - nin)
        n_cur = nin
        prefix = lax.convert_element_type(bsel, jnp.uint32)

        @plsc.parallel_loop(0, D, step=_L, unroll=4, carry=jnp.int32(0))
        def scat2(i, off, prefix=prefix):
            s = pl.ds(i, _L)
            u = ubuf[s]
            msk = (u >> np.uint32(24)) == prefix
            dig = lax.convert_element_type(
                (u >> np.uint32(16)) & np.uint32(0xFF), jnp.int32)
            plsc.addupdate_scatter(hist, [laneoff + dig], ones, mask=msk)
            plsc.store_compressed(cbuf.at[pl.ds(off, _L)], u, mask=msk)
            return off + jnp.sum(msk.astype(jnp.int32))

        n1_s = jnp.max(nin)
        n1_v = nin

        bsel, nin, nbelow = scan_pass(n_cur - k_cur)
        k_cur = k_cur - (n_cur - nbelow - nin)
        n_cur = nin
        prefix = (prefix << np.uint32(8)) | lax.convert_element_type(
            bsel, jnp.uint32)

        @plsc.parallel_loop(0, ((n1_s + _L - 1) // _L) * _L, step=_L)
        def _(j, prefix=prefix, n1_v=n1_v):
            s = pl.ds(j, _L)
            u = cbuf[s]
            valid = (laneseq + j) < n1_v
            msk = jnp.logical_and(valid, (u >> np.uint32(16)) == prefix)
            dig = lax.convert_element_type(
                (u >> np.uint32(8)) & np.uint32(0xFF), jnp.int32)
            plsc.addupdate_scatter(hist, [laneoff + dig], ones, mask=msk)

        bsel, nin, nbelow = scan_pass(n_cur - k_cur)
        k_cur = k_cur - (n_cur - nbelow - nin)
        n_cur = nin
        prefix = (prefix << np.uint32(8)) | lax.convert_element_type(
            bsel, jnp.uint32)

        @plsc.parallel_loop(0, ((n1_s + _L - 1) // _L) * _L, step=_L)
        def _(j, prefix=prefix, n1_v=n1_v):
            s = pl.ds(j, _L)
            u = cbuf[s]
            valid = (laneseq + j) < n1_v
            msk = jnp.logical_and(valid, (u >> np.uint32(8)) == prefix)
            dig = lax.convert_element_type(u & np.uint32(0xFF), jnp.int32)
            plsc.addupdate_scatter(hist, [laneoff + dig], ones, mask=msk)

        bsel, _, _ = scan_pass(n_cur - k_cur)
        thresh = (prefix << np.uint32(8)) | lax.convert_element_type(
            bsel, jnp.uint32)

        @plsc.parallel_loop(0, D, step=_L, unroll=4)
        def _(i, thresh=thresh):
            s = pl.ds(i, _L)
            u = ubuf[s]
            p = fbuf[s] * dbuf[s]
            fbuf[s] = jnp.where(u >= thresh, p, jnp.float32(0.0))

        pltpu.sync_copy(fbuf, out_hbm.at[row])
        return 0

    lax.fori_loop(0, rows_per_w, row_step, 0)


def kernel(f_x, dead_latents):
    B, D = f_x.shape
    mesh = plsc.VectorSubcoreMesh(core_axis_name="c", subcore_axis_name="s",
                                  num_cores=_NC, num_subcores=_NS)
    run = pl.kernel(
        _sc_body,
        out_type=jax.ShapeDtypeStruct((B, D), jnp.float32),
        mesh=mesh,
        compiler_params=pltpu.CompilerParams(needs_layout_passes=False),
        scratch_types=[
            pltpu.VMEM((D,), jnp.float32),
            pltpu.VMEM((D,), jnp.float32),
            pltpu.VMEM((D,), jnp.uint32),
            pltpu.VMEM((D + _L,), jnp.uint32),
            pltpu.VMEM((_HIST,), jnp.int32),
        ],
    )
    return run(f_x, dead_latents)
